# baseline (device time: 37689 ns/iter reference)
import jax
import jax.numpy as jnp
from jax import lax
from jax.experimental import pallas as pl
from jax.experimental.pallas import tpu as pltpu

N_DEV = 4
QROWS = 256
CHUNKS = ((0, 768), (768, 256))
WMAX = 768


def kernel(x, W1, W2):
    m, _ = x.shape
    k_h = W1.shape[1]
    _, n_out = W2.shape

    def body(x_ref, w1_ref, w2_ref, out_ref, h_ref, recv_buf,
             send_sems, recv_sems):
        d = lax.axis_index("i")
        pA = jnp.bitwise_xor(d, 1)
        pB = 3 - d
        kb1 = jnp.where((d == 1) | (d == 2), 1, 0)
        kb2 = d // 2
        K = [kb1, 2 + kb2]
        S = [1 - kb1, 3 - kb2]
        partners = [[pA, pB], [pB, pA], [pA, pB]]

        barrier_sem = pltpu.get_barrier_semaphore()
        for nbr in (pA, pB):
            pl.semaphore_signal(
                barrier_sem, inc=1,
                device_id=(nbr,), device_id_type=pl.DeviceIdType.MESH,
            )
        pl.semaphore_wait(barrier_sem, 2)

        w1b = w1_ref[...].astype(jnp.bfloat16)
        w2b = w2_ref[...].astype(jnp.bfloat16)

        def rows(qi):
            return pl.ds(qi * QROWS, QROWS)

        def cols(c):
            o, w = CHUNKS[c]
            return pl.ds(o, w)

        def compute_h(qi):
            xq = x_ref[rows(qi), :].astype(jnp.bfloat16)
            h_ref[rows(qi), :] = jnp.maximum(
                jnp.dot(xq, w1b, preferred_element_type=jnp.float32), 0.0
            ).astype(jnp.bfloat16)

        def compute_p(qi, c):
            o, w = CHUNKS[c]
            return jnp.dot(
                h_ref[rows(qi), :], w2b[:, o:o + w],
                preferred_element_type=jnp.float32).astype(jnp.bfloat16)

        rdmas = {}

        def send(c, s, b, src_q, dst_q):
            w = CHUNKS[c][1]
            if s == 2:
                dst = out_ref.at[rows(dst_q), cols(c)]
            else:
                dst = recv_buf.at[c, s, b, :, :w]
            r = pltpu.make_async_remote_copy(
                src_ref=out_ref.at[rows(src_q), cols(c)],
                dst_ref=dst,
                send_sem=send_sems.at[c, s, b],
                recv_sem=recv_sems.at[c, s, b],
                device_id=(partners[s][b],),
                device_id_type=pl.DeviceIdType.MESH,
            )
            r.start()
            rdmas[(c, s, b)] = r

        def reduce_and_send(c, s, b):
            w = CHUNKS[c][1]
            rdmas[(c, s, b)].wait()
            acc = (out_ref[rows(K[b]), cols(c)].astype(jnp.float32)
                   + recv_buf[c, s, b, :, :w].astype(jnp.float32))
            out_ref[rows(K[b]), cols(c)] = acc.astype(jnp.bfloat16)
            send(c, s + 1, b, K[b], K[b])

        for b in range(2):
            compute_h(S[b])
            out_ref[rows(S[b]), cols(0)] = compute_p(S[b], 0)
            send(0, 0, b, S[b], K[b])
        for b in range(2):
            out_ref[rows(S[b]), cols(1)] = compute_p(S[b], 1)
            send(1, 0, b, S[b], K[b])

        for b in range(2):
            compute_h(K[b])
            out_ref[rows(K[b]), cols(0)] = compute_p(K[b], 0)
        for b in range(2):
            reduce_and_send(0, 0, b)

        for b in range(2):
            out_ref[rows(K[b]), cols(1)] = compute_p(K[b], 1)
        for b in range(2):
            reduce_and_send(1, 0, b)

        for c in range(2):
            for b in range(2):
                reduce_and_send(c, 1, b)

        for c in range(2):
            for b in range(2):
                rdmas[(c, 2, b)].wait()

    return pl.pallas_call(
        body,
        out_shape=jax.ShapeDtypeStruct((m, n_out), jnp.bfloat16),
        in_specs=[
            pl.BlockSpec(memory_space=pltpu.VMEM),
            pl.BlockSpec(memory_space=pltpu.VMEM),
            pl.BlockSpec(memory_space=pltpu.VMEM),
        ],
        out_specs=pl.BlockSpec(memory_space=pltpu.VMEM),
        scratch_shapes=[
            pltpu.VMEM((m, k_h), jnp.bfloat16),
            pltpu.VMEM((2, 2, 2, QROWS, WMAX), jnp.bfloat16),
            pltpu.SemaphoreType.DMA((2, 3, 2)),
            pltpu.SemaphoreType.DMA((2, 3, 2)),
        ],
        compiler_params=pltpu.CompilerParams(collective_id=0),
    )(x, W1, W2)


# device time: 32218 ns/iter; 1.1698x vs baseline; 1.1698x over previous
import jax
import jax.numpy as jnp
from jax import lax
from jax.experimental import pallas as pl
from jax.experimental.pallas import tpu as pltpu

N_DEV = 4
QROWS = 256
HCOLS = 512


def kernel(x, W1, W2):
    m, _ = x.shape
    k_h = W1.shape[1]
    _, n_out = W2.shape

    def body(x_ref, w1_ref, w2_ref, out_ref, h_ref, recv_buf,
             send_sems, recv_sems):
        d = lax.axis_index("i")
        pA = jnp.bitwise_xor(d, 1)
        pB = 3 - d
        kb1 = jnp.where((d == 1) | (d == 2), 1, 0)
        kb2 = d // 2
        K = [kb1, 2 + kb2]
        S = [1 - kb1, 3 - kb2]
        partners = [[pA, pB], [pB, pA], [pA, pB]]

        barrier_sem = pltpu.get_barrier_semaphore()
        for nbr in (pA, pB):
            pl.semaphore_signal(
                barrier_sem, inc=1,
                device_id=(nbr,), device_id_type=pl.DeviceIdType.MESH,
            )
        pl.semaphore_wait(barrier_sem, 2)

        w1b = w1_ref[...].astype(jnp.bfloat16)
        w2b = w2_ref[...].astype(jnp.bfloat16)

        def rows(qi):
            return pl.ds(qi * QROWS, QROWS)

        def cols(c):
            return pl.ds(c * HCOLS, HCOLS)

        def compute_h(qi):
            xq = x_ref[rows(qi), :].astype(jnp.bfloat16)
            h_ref[rows(qi), :] = jnp.maximum(
                jnp.dot(xq, w1b, preferred_element_type=jnp.float32), 0.0
            ).astype(jnp.bfloat16)

        def compute_p(qi, c):
            return jnp.dot(
                h_ref[rows(qi), :], w2b[:, c * HCOLS:(c + 1) * HCOLS],
                preferred_element_type=jnp.float32).astype(jnp.bfloat16)

        rdmas = {}

        def send(c, s, b, src_q, dst_q):
            if s == 2:
                dst = out_ref.at[rows(dst_q), cols(c)]
            else:
                dst = recv_buf.at[c, s, b]
            r = pltpu.make_async_remote_copy(
                src_ref=out_ref.at[rows(src_q), cols(c)],
                dst_ref=dst,
                send_sem=send_sems.at[c, s, b],
                recv_sem=recv_sems.at[c, s, b],
                device_id=(partners[s][b],),
                device_id_type=pl.DeviceIdType.MESH,
            )
            r.start()
            rdmas[(c, s, b)] = r

        def reduce_and_send(c, s, b):
            rdmas[(c, s, b)].wait()
            acc = (out_ref[rows(K[b]), cols(c)].astype(jnp.float32)
                   + recv_buf[c, s, b].astype(jnp.float32))
            out_ref[rows(K[b]), cols(c)] = acc.astype(jnp.bfloat16)
            send(c, s + 1, b, K[b], K[b])

        for b in range(2):
            compute_h(S[b])
            out_ref[rows(S[b]), cols(0)] = compute_p(S[b], 0)
            send(0, 0, b, S[b], K[b])
        for b in range(2):
            out_ref[rows(S[b]), cols(1)] = compute_p(S[b], 1)
            send(1, 0, b, S[b], K[b])

        for b in range(2):
            compute_h(K[b])
            out_ref[rows(K[b]), cols(0)] = compute_p(K[b], 0)
        for b in range(2):
            reduce_and_send(0, 0, b)

        for b in range(2):
            out_ref[rows(K[b]), cols(1)] = compute_p(K[b], 1)
        for b in range(2):
            reduce_and_send(1, 0, b)

        for c in range(2):
            for b in range(2):
                reduce_and_send(c, 1, b)

        for c in range(2):
            for b in range(2):
                rdmas[(c, 2, b)].wait()

    return pl.pallas_call(
        body,
        out_shape=jax.ShapeDtypeStruct((m, n_out), jnp.bfloat16),
        in_specs=[
            pl.BlockSpec(memory_space=pltpu.VMEM),
            pl.BlockSpec(memory_space=pltpu.VMEM),
            pl.BlockSpec(memory_space=pltpu.VMEM),
        ],
        out_specs=pl.BlockSpec(memory_space=pltpu.VMEM),
        scratch_shapes=[
            pltpu.VMEM((m, k_h), jnp.bfloat16),
            pltpu.VMEM((2, 2, 2, QROWS, HCOLS), jnp.bfloat16),
            pltpu.SemaphoreType.DMA((2, 3, 2)),
            pltpu.SemaphoreType.DMA((2, 3, 2)),
        ],
        compiler_params=pltpu.CompilerParams(collective_id=0),
    )(x, W1, W2)
